# baseline (speedup 1.0000x reference)
import functools

import jax
import jax.numpy as jnp
from jax import lax
from jax.experimental import pallas as pl
from jax.experimental.pallas import tpu as pltpu
from jax.experimental.pallas import tpu_sc as plsc

B = 16384
DIM = 32
CHUNK = 128
PACK = 4
WPR = DIM * PACK // 2


def kernel(users, items, user_latent, item_latent, user_bias, item_bias):
    info = plsc.get_sparse_core_info()
    nc, ns = info.num_cores, info.num_subcores
    nw = nc * ns
    bpw = B // nw
    nchunk = bpw // CHUNK

    mesh = plsc.VectorSubcoreMesh(core_axis_name="c", subcore_axis_name="s")

    @functools.partial(
        pl.kernel,
        out_type=jax.ShapeDtypeStruct((B,), jnp.float32),
        mesh=mesh,
        compiler_params=pltpu.CompilerParams(needs_layout_passes=False,
                                             use_tc_tiling_on_sc=False),
        scratch_types=[
            pltpu.VMEM((nchunk, CHUNK), jnp.int32),
            pltpu.VMEM((nchunk, CHUNK), jnp.int32),
            pltpu.VMEM((nchunk, CHUNK), jnp.int32),
            pltpu.VMEM((nchunk, CHUNK), jnp.int32),
            pltpu.VMEM((2, CHUNK, WPR), jnp.int32),
            pltpu.VMEM((2, CHUNK, WPR), jnp.int32),
            pltpu.VMEM((bpw,), jnp.float32),
            pltpu.VMEM((bpw,), jnp.float32),
            pltpu.VMEM((bpw,), jnp.float32),
            pltpu.SemaphoreType.DMA,
            pltpu.SemaphoreType.DMA,
            pltpu.SemaphoreType.DMA,
        ],
    )
    def mf_kernel(users_hbm, items_hbm, ulp_hbm, ilp_hbm, ub_hbm, ib_hbm,
                  out_hbm, uidx_v, iidx_v, upr_v, ipr_v, urows_v, irows_v,
                  ub_v, ib_v, out_v, sem_b, sem0, sem1):
        wid = lax.axis_index("s") * nc + lax.axis_index("c")
        base = wid * bpw

        for j in range(nchunk):
            pltpu.sync_copy(users_hbm.at[pl.ds(base + j * CHUNK, CHUNK)],
                            uidx_v.at[j])
            pltpu.sync_copy(items_hbm.at[pl.ds(base + j * CHUNK, CHUNK)],
                            iidx_v.at[j])

        bias_copies = []
        for j in range(nchunk):
            sl = pl.ds(j * CHUNK, CHUNK)
            bias_copies.append(pltpu.async_copy(ub_hbm.at[uidx_v.at[j]],
                                                ub_v.at[sl], sem_b))
            bias_copies.append(pltpu.async_copy(ib_hbm.at[iidx_v.at[j]],
                                                ib_v.at[sl], sem_b))

        def pbody(k, carry):
            for j in range(nchunk):
                sl = pl.ds(k * 16, 16)
                upr_v[j, sl] = lax.shift_right_logical(uidx_v[j, sl], 2)
                ipr_v[j, sl] = lax.shift_right_logical(iidx_v[j, sl], 2)
            return carry

        lax.fori_loop(0, CHUNK // 16, pbody, 0)

        sems = (sem0, sem1)

        def fire(j, buf):
            pltpu.async_copy(ulp_hbm.at[upr_v.at[j]], urows_v.at[buf],
                             sems[buf])
            pltpu.async_copy(ilp_hbm.at[ipr_v.at[j]], irows_v.at[buf],
                             sems[buf])

        def drain(j, buf):
            pltpu.make_async_copy(ulp_hbm.at[upr_v.at[j]], urows_v.at[buf],
                                  sems[buf]).wait()
            pltpu.make_async_copy(ilp_hbm.at[ipr_v.at[j]], irows_v.at[buf],
                                  sems[buf]).wait()

        fire(0, 0)
        for c in bias_copies:
            c.wait()

        lane16 = lax.iota(jnp.int32, 16)

        def make_cbody(j, buf):
            def cbody(g, carry):
                s = g * 16
                un = uidx_v[j, pl.ds(s, 16)]
                im = iidx_v[j, pl.ds(s, 16)]
                urow = s + lane16
                ucol = lax.shift_left(un & 3, 4)
                icol = lax.shift_left(im & 3, 4)
                acc = (ub_v[pl.ds(j * CHUNK + s, 16)]
                       + ib_v[pl.ds(j * CHUNK + s, 16)])
                for dd in range(DIM // 2):
                    uw = plsc.load_gather(urows_v.at[buf], [urow, ucol + dd])
                    iw = plsc.load_gather(irows_v.at[buf], [urow, icol + dd])
                    ua, ub2 = plsc.unpack(
                        plsc.bitcast(uw, jnp.bfloat16),
                        format=plsc.PackFormat.INTERLEAVED)
                    ia, ib2 = plsc.unpack(
                        plsc.bitcast(iw, jnp.bfloat16),
                        format=plsc.PackFormat.INTERLEAVED)
                    acc = acc + ua * ia + ub2 * ib2
                out_v[pl.ds(j * CHUNK + s, 16)] = acc
                return carry
            return cbody

        for j in range(nchunk):
            buf = j % 2
            if j + 1 < nchunk:
                fire(j + 1, 1 - buf)
            drain(j, buf)
            lax.fori_loop(0, CHUNK // 16, make_cbody(j, buf), 0)

        pltpu.sync_copy(out_v, out_hbm.at[pl.ds(base, bpw)])

    n_packed = user_latent.shape[0] // PACK
    ulp = lax.bitcast_convert_type(
        user_latent.astype(jnp.bfloat16).reshape(n_packed, WPR, 2),
        jnp.int32)
    ilp = lax.bitcast_convert_type(
        item_latent.astype(jnp.bfloat16).reshape(n_packed, WPR, 2),
        jnp.int32)
    return mf_kernel(users, items, ulp, ilp,
                     user_bias.T.reshape(-1), item_bias.T.reshape(-1))


# speedup vs baseline: 3.6751x; 3.6751x over previous
import functools

import jax
import jax.numpy as jnp
from jax import lax
from jax.experimental import pallas as pl
from jax.experimental.pallas import tpu as pltpu
from jax.experimental.pallas import tpu_sc as plsc

B = 16384
DIM = 32
CHUNK = 128


def kernel(users, items, user_latent, item_latent, user_bias, item_bias):
    info = plsc.get_sparse_core_info()
    nc, ns = info.num_cores, info.num_subcores
    nw = nc * ns
    bpw = B // nw
    nchunk = bpw // CHUNK

    mesh = plsc.VectorSubcoreMesh(core_axis_name="c", subcore_axis_name="s")

    @functools.partial(
        pl.kernel,
        out_type=jax.ShapeDtypeStruct((B,), jnp.float32),
        mesh=mesh,
        compiler_params=pltpu.CompilerParams(needs_layout_passes=False,
                                             use_tc_tiling_on_sc=False),
        scratch_types=[
            pltpu.VMEM((nchunk, CHUNK), jnp.int32),
            pltpu.VMEM((nchunk, CHUNK), jnp.int32),
            pltpu.VMEM((DIM * bpw,), jnp.float32),
            pltpu.VMEM((DIM * bpw,), jnp.float32),
            pltpu.VMEM((bpw,), jnp.float32),
            pltpu.VMEM((bpw,), jnp.float32),
            pltpu.VMEM((bpw,), jnp.float32),
            pltpu.SemaphoreType.DMA,
        ],
    )
    def mf_kernel(users_hbm, items_hbm, ult_hbm, ilt_hbm, ub_hbm, ib_hbm,
                  out_hbm, uidx_v, iidx_v, ut_v, it_v, ub_v, ib_v,
                  out_v, sem):
        wid = lax.axis_index("s") * nc + lax.axis_index("c")
        base = wid * bpw

        for j in range(nchunk):
            pltpu.sync_copy(users_hbm.at[pl.ds(base + j * CHUNK, CHUNK)],
                            uidx_v.at[j])
            pltpu.sync_copy(items_hbm.at[pl.ds(base + j * CHUNK, CHUNK)],
                            iidx_v.at[j])

        copies = []
        for j in range(nchunk):
            sl = pl.ds(j * CHUNK, CHUNK)
            copies.append(pltpu.async_copy(ub_hbm.at[uidx_v.at[j]],
                                           ub_v.at[sl], sem))
            copies.append(pltpu.async_copy(ib_hbm.at[iidx_v.at[j]],
                                           ib_v.at[sl], sem))
        for d in range(DIM):
            for j in range(nchunk):
                dsl = pl.ds(d * bpw + j * CHUNK, CHUNK)
                copies.append(pltpu.async_copy(
                    ult_hbm.at[d].at[uidx_v.at[j]], ut_v.at[dsl], sem))
                copies.append(pltpu.async_copy(
                    ilt_hbm.at[d].at[iidx_v.at[j]], it_v.at[dsl], sem))
        for c in copies:
            c.wait()

        def gbody(g, carry):
            s = g * 16
            acc = ub_v[pl.ds(s, 16)] + ib_v[pl.ds(s, 16)]
            for d in range(DIM):
                acc = acc + (ut_v[pl.ds(d * bpw + s, 16)]
                             * it_v[pl.ds(d * bpw + s, 16)])
            out_v[pl.ds(s, 16)] = acc
            return carry

        lax.fori_loop(0, bpw // 16, gbody, 0)

        pltpu.sync_copy(out_v, out_hbm.at[pl.ds(base, bpw)])

    one = lax.optimization_barrier(jnp.float32(1.0))
    return mf_kernel(users, items, user_latent.T * one, item_latent.T * one,
                     user_bias.T.reshape(-1), item_bias.T.reshape(-1))


# speedup vs baseline: 21.6591x; 5.8935x over previous
import functools

import jax
import jax.numpy as jnp
from jax import lax
from jax.experimental import pallas as pl
from jax.experimental.pallas import tpu as pltpu
from jax.experimental.pallas import tpu_sc as plsc

B = 16384
DIM = 32
CHUNK = 128
PACK = 2


def kernel(users, items, user_latent, item_latent, user_bias, item_bias):
    info = plsc.get_sparse_core_info()
    nc, ns = info.num_cores, info.num_subcores
    nw = nc * ns
    bpw = B // nw
    nchunk = bpw // CHUNK

    mesh = plsc.VectorSubcoreMesh(core_axis_name="c", subcore_axis_name="s")

    @functools.partial(
        pl.kernel,
        out_type=jax.ShapeDtypeStruct((B,), jnp.float32),
        mesh=mesh,
        compiler_params=pltpu.CompilerParams(needs_layout_passes=False,
                                             use_tc_tiling_on_sc=False),
        scratch_types=[
            pltpu.VMEM((nchunk, CHUNK), jnp.int32),
            pltpu.VMEM((nchunk, CHUNK), jnp.int32),
            pltpu.VMEM((nchunk, CHUNK), jnp.int32),
            pltpu.VMEM((nchunk, CHUNK), jnp.int32),
            pltpu.VMEM((2, CHUNK, 64), jnp.float32),
            pltpu.VMEM((2, CHUNK, 64), jnp.float32),
            pltpu.VMEM((bpw,), jnp.float32),
            pltpu.VMEM((bpw,), jnp.float32),
            pltpu.VMEM((bpw,), jnp.float32),
            pltpu.SemaphoreType.DMA,
            pltpu.SemaphoreType.DMA,
            pltpu.SemaphoreType.DMA,
        ],
    )
    def mf_kernel(users_hbm, items_hbm, ulp_hbm, ilp_hbm, ub_hbm, ib_hbm,
                  out_hbm, uidx_v, iidx_v, upr_v, ipr_v, urows_v, irows_v,
                  ub_v, ib_v, out_v, sem_b, sem0, sem1):
        wid = lax.axis_index("s") * nc + lax.axis_index("c")
        base = wid * bpw

        for j in range(nchunk):
            pltpu.sync_copy(users_hbm.at[pl.ds(base + j * CHUNK, CHUNK)],
                            uidx_v.at[j])
            pltpu.sync_copy(items_hbm.at[pl.ds(base + j * CHUNK, CHUNK)],
                            iidx_v.at[j])

        bias_copies = []
        for j in range(nchunk):
            sl = pl.ds(j * CHUNK, CHUNK)
            bias_copies.append(pltpu.async_copy(ub_hbm.at[uidx_v.at[j]],
                                                ub_v.at[sl], sem_b))
            bias_copies.append(pltpu.async_copy(ib_hbm.at[iidx_v.at[j]],
                                                ib_v.at[sl], sem_b))

        def pbody(k, carry):
            for j in range(nchunk):
                sl = pl.ds(k * 16, 16)
                upr_v[j, sl] = lax.shift_right_logical(uidx_v[j, sl], 1)
                ipr_v[j, sl] = lax.shift_right_logical(iidx_v[j, sl], 1)
            return carry

        lax.fori_loop(0, CHUNK // 16, pbody, 0)

        sems = (sem0, sem1)

        def fire(j, buf):
            pltpu.async_copy(ulp_hbm.at[upr_v.at[j]], urows_v.at[buf],
                             sems[buf])
            pltpu.async_copy(ilp_hbm.at[ipr_v.at[j]], irows_v.at[buf],
                             sems[buf])

        def drain(j, buf):
            pltpu.make_async_copy(ulp_hbm.at[upr_v.at[j]], urows_v.at[buf],
                                  sems[buf]).wait()
            pltpu.make_async_copy(ilp_hbm.at[ipr_v.at[j]], irows_v.at[buf],
                                  sems[buf]).wait()

        fire(0, 0)
        for c in bias_copies:
            c.wait()

        lane16 = lax.iota(jnp.int32, 16)

        def make_cbody(j, buf):
            def cbody(g, carry):
                s = g * 16
                un = uidx_v[j, pl.ds(s, 16)]
                im = iidx_v[j, pl.ds(s, 16)]
                urow = s + lane16
                ucol = lax.shift_left(un & 1, 5)
                icol = lax.shift_left(im & 1, 5)
                acc = (ub_v[pl.ds(j * CHUNK + s, 16)]
                       + ib_v[pl.ds(j * CHUNK + s, 16)])
                for d in range(DIM):
                    acc = acc + (plsc.load_gather(urows_v.at[buf],
                                                  [urow, ucol + d])
                                 * plsc.load_gather(irows_v.at[buf],
                                                    [urow, icol + d]))
                out_v[pl.ds(j * CHUNK + s, 16)] = acc
                return carry
            return cbody

        for j in range(nchunk):
            buf = j % 2
            if j + 1 < nchunk:
                fire(j + 1, 1 - buf)
            drain(j, buf)
            lax.fori_loop(0, CHUNK // 16, make_cbody(j, buf), 0)

        pltpu.sync_copy(out_v, out_hbm.at[pl.ds(base, bpw)])

    ulp = user_latent.reshape(user_latent.shape[0] // PACK, 64)
    ilp = item_latent.reshape(item_latent.shape[0] // PACK, 64)
    return mf_kernel(users, items, ulp, ilp,
                     user_bias.T.reshape(-1), item_bias.T.reshape(-1))
